# baseline (speedup 1.0000x reference)
import functools

import jax
import jax.numpy as jnp
from jax import lax
from jax.experimental import pallas as pl
from jax.experimental.pallas import tpu as pltpu
from jax.experimental.pallas import tpu_sc as plsc

_NC = 2
_NS = 16
_NW = _NC * _NS
_LANES = 16


@functools.partial(jax.jit, static_argnames=("batch",))
def _irt_sc(skills, ability16, packed, *, batch):
    b_per_w = batch // _NW
    half = b_per_w // 2
    mesh = plsc.VectorSubcoreMesh(
        core_axis_name="c", subcore_axis_name="s",
        num_cores=_NC, num_subcores=_NS)

    @functools.partial(
        pl.kernel,
        out_type=jax.ShapeDtypeStruct((batch,), jnp.float32),
        mesh=mesh,
        scratch_types=[
            pltpu.VMEM((b_per_w,), jnp.int32),
            pltpu.VMEM((b_per_w,), jnp.int32),
            pltpu.VMEM((b_per_w,), jnp.int32),
            pltpu.VMEM((b_per_w,), jnp.float32),
            pltpu.VMEM((b_per_w,), jnp.float32),
            pltpu.VMEM((_LANES,), jnp.float32),
            pltpu.SemaphoreType.DMA,
            pltpu.SemaphoreType.DMA,
            pltpu.SemaphoreType.DMA,
        ],
    )
    def k(skills_hbm, ab_hbm, packed_hbm, out_hbm,
          idx_v, idxd_v, idxc_v, diff_v, disc_v, ab_v, sem0, sem1, semo):
        wid = lax.axis_index("s") * _NC + lax.axis_index("c")
        base = wid * b_per_w
        lo = pl.ds(0, half)
        hi = pl.ds(half, half)
        pltpu.sync_copy(skills_hbm.at[pl.ds(base, half)], idx_v.at[lo])
        for i in range(half // _LANES):
            sl = pl.ds(i * _LANES, _LANES)
            two = idx_v[sl] * 2
            idxd_v[sl] = two
            idxc_v[sl] = two + 1
        cp0 = [
            pltpu.async_copy(packed_hbm.at[idxd_v.at[lo]], diff_v.at[lo], sem0),
            pltpu.async_copy(packed_hbm.at[idxc_v.at[lo]], disc_v.at[lo], sem0),
        ]
        pltpu.sync_copy(skills_hbm.at[pl.ds(base + half, half)], idx_v.at[hi])
        for i in range(half // _LANES, b_per_w // _LANES):
            sl = pl.ds(i * _LANES, _LANES)
            two = idx_v[sl] * 2
            idxd_v[sl] = two
            idxc_v[sl] = two + 1
        cp1 = [
            pltpu.async_copy(packed_hbm.at[idxd_v.at[hi]], diff_v.at[hi], sem1),
            pltpu.async_copy(packed_hbm.at[idxc_v.at[hi]], disc_v.at[hi], sem1),
        ]
        pltpu.sync_copy(ab_hbm, ab_v)
        a = ab_v[:]
        for cp in cp0:
            cp.wait()
        for i in range(half // _LANES):
            sl = pl.ds(i * _LANES, _LANES)
            x = disc_v[sl] * (a - diff_v[sl])
            diff_v[sl] = 1.0 / (1.0 + jnp.exp(-x))
        wr0 = pltpu.async_copy(
            diff_v.at[lo], out_hbm.at[pl.ds(base, half)], semo)
        for cp in cp1:
            cp.wait()
        for i in range(half // _LANES, b_per_w // _LANES):
            sl = pl.ds(i * _LANES, _LANES)
            x = disc_v[sl] * (a - diff_v[sl])
            diff_v[sl] = 1.0 / (1.0 + jnp.exp(-x))
        wr1 = pltpu.async_copy(
            diff_v.at[hi], out_hbm.at[pl.ds(base + half, half)], semo)
        wr0.wait()
        wr1.wait()

    return k(skills, ability16, packed)


def kernel(skills, ability_table, difficulty_table, discrimination_table):
    batch = skills.shape[0]
    if skills.dtype != jnp.int32:
        skills = skills.astype(jnp.int32)
    ability16 = jnp.broadcast_to(ability_table.reshape(()), (_LANES,))
    packed = jnp.concatenate(
        [difficulty_table, discrimination_table], axis=1).reshape(-1)
    out = _irt_sc(skills, ability16, packed, batch=batch)
    return out.reshape(batch, 1)


# speedup vs baseline: 2.9721x; 2.9721x over previous
import functools

import jax
import jax.numpy as jnp
from jax import lax
from jax.experimental import pallas as pl
from jax.experimental.pallas import tpu as pltpu
from jax.experimental.pallas import tpu_sc as plsc

_NC = 2
_NS = 16
_NW = _NC * _NS
_LANES = 16

_MESH = dict(core_axis_name="c", subcore_axis_name="s",
             num_cores=_NC, num_subcores=_NS)


@functools.partial(jax.jit, static_argnames=("batch",))
def _irt_sc(skills, ability16, difficulty, discrimination, *, batch):
    b_per_w = batch // _NW
    half = b_per_w // 2

    @functools.partial(
        pl.kernel,
        out_type=jax.ShapeDtypeStruct((batch,), jnp.float32),
        mesh=plsc.VectorSubcoreMesh(**_MESH),
        scratch_types=[
            pltpu.VMEM((b_per_w,), jnp.int32),
            pltpu.VMEM((b_per_w,), jnp.float32),
            pltpu.SemaphoreType.DMA,
        ],
    )
    def gather_disc(skills_hbm, disc_hbm, out_hbm, idx_v, disc_v, sem):
        wid = lax.axis_index("s") * _NC + lax.axis_index("c")
        base = wid * b_per_w
        pltpu.sync_copy(skills_hbm.at[pl.ds(base, b_per_w)], idx_v)
        pltpu.async_copy(disc_hbm.at[idx_v], disc_v, sem).wait()
        pltpu.sync_copy(disc_v, out_hbm.at[pl.ds(base, b_per_w)])

    @functools.partial(
        pl.kernel,
        out_type=jax.ShapeDtypeStruct((batch,), jnp.float32),
        mesh=plsc.VectorSubcoreMesh(**_MESH),
        scratch_types=[
            pltpu.VMEM((b_per_w,), jnp.int32),
            pltpu.VMEM((b_per_w,), jnp.float32),
            pltpu.VMEM((b_per_w,), jnp.float32),
            pltpu.VMEM((_LANES,), jnp.float32),
            pltpu.SemaphoreType.DMA,
            pltpu.SemaphoreType.DMA,
            pltpu.SemaphoreType.DMA,
        ],
    )
    def diff_and_sigmoid(skills_hbm, diff_hbm, disc_g_hbm, ab_hbm, out_hbm,
                         idx_v, diff_v, disc_v, ab_v, sem0, sem1, semo):
        wid = lax.axis_index("s") * _NC + lax.axis_index("c")
        base = wid * b_per_w
        lo = pl.ds(0, half)
        hi = pl.ds(half, half)
        pltpu.sync_copy(skills_hbm.at[pl.ds(base, b_per_w)], idx_v)
        cp0 = pltpu.async_copy(diff_hbm.at[idx_v.at[lo]], diff_v.at[lo], sem0)
        cp1 = pltpu.async_copy(diff_hbm.at[idx_v.at[hi]], diff_v.at[hi], sem1)
        cpd = pltpu.async_copy(
            disc_g_hbm.at[pl.ds(base, b_per_w)], disc_v, semo)
        pltpu.sync_copy(ab_hbm, ab_v)
        a = ab_v[:]
        cpd.wait()
        cp0.wait()
        for i in range(half // _LANES):
            sl = pl.ds(i * _LANES, _LANES)
            x = disc_v[sl] * (a - diff_v[sl])
            diff_v[sl] = 1.0 / (1.0 + jnp.exp(-x))
        wr0 = pltpu.async_copy(
            diff_v.at[lo], out_hbm.at[pl.ds(base, half)], semo)
        cp1.wait()
        for i in range(half // _LANES, b_per_w // _LANES):
            sl = pl.ds(i * _LANES, _LANES)
            x = disc_v[sl] * (a - diff_v[sl])
            diff_v[sl] = 1.0 / (1.0 + jnp.exp(-x))
        wr1 = pltpu.async_copy(
            diff_v.at[hi], out_hbm.at[pl.ds(base + half, half)], semo)
        wr0.wait()
        wr1.wait()

    disc_g = gather_disc(skills, discrimination)
    return diff_and_sigmoid(skills, difficulty, disc_g, ability16)


def kernel(skills, ability_table, difficulty_table, discrimination_table):
    batch = skills.shape[0]
    if skills.dtype != jnp.int32:
        skills = skills.astype(jnp.int32)
    ability16 = jnp.broadcast_to(ability_table.reshape(()), (_LANES,))
    diff = difficulty_table.reshape(-1)
    disc = discrimination_table.reshape(-1)
    out = _irt_sc(skills, ability16, diff, disc, batch=batch)
    return out.reshape(batch, 1)


# speedup vs baseline: 3.2152x; 1.0818x over previous
import functools

import jax
import jax.numpy as jnp
from jax import lax
from jax.experimental import pallas as pl
from jax.experimental.pallas import tpu as pltpu
from jax.experimental.pallas import tpu_sc as plsc

_NC = 2
_NS = 16
_NW = _NC * _NS
_LANES = 16
_NQ = 4


@functools.partial(jax.jit, static_argnames=("batch",))
def _irt_sc(skills, ability16, difficulty, discrimination, *, batch):
    b_per_w = batch // _NW
    quarter = b_per_w // _NQ
    mesh = plsc.VectorSubcoreMesh(
        core_axis_name="c", subcore_axis_name="s",
        num_cores=_NC, num_subcores=_NS)

    @functools.partial(
        pl.kernel,
        out_type=jax.ShapeDtypeStruct((batch,), jnp.float32),
        mesh=mesh,
        scratch_types=[
            pltpu.VMEM((b_per_w,), jnp.int32),
            pltpu.VMEM((b_per_w,), jnp.float32),
            pltpu.VMEM((b_per_w,), jnp.float32),
            pltpu.VMEM((_LANES,), jnp.float32),
            [pltpu.SemaphoreType.DMA] * _NQ,
            pltpu.SemaphoreType.DMA,
        ],
    )
    def k(skills_hbm, ab_hbm, diff_hbm, disc_hbm, out_hbm,
          idx_v, diff_v, disc_v, ab_v, sems, semo):
        wid = lax.axis_index("s") * _NC + lax.axis_index("c")
        base = wid * b_per_w
        pltpu.sync_copy(skills_hbm.at[pl.ds(base, b_per_w)], idx_v)
        cps = []
        for q in range(_NQ):
            ds = pl.ds(q * quarter, quarter)
            cps.append((
                pltpu.async_copy(diff_hbm.at[idx_v.at[ds]], diff_v.at[ds], sems[q]),
                pltpu.async_copy(disc_hbm.at[idx_v.at[ds]], disc_v.at[ds], sems[q]),
            ))
        pltpu.sync_copy(ab_hbm, ab_v)
        a = ab_v[:]
        writes = []
        for q in range(_NQ):
            for cp in cps[q]:
                cp.wait()
            for i in range(q * quarter // _LANES, (q + 1) * quarter // _LANES):
                sl = pl.ds(i * _LANES, _LANES)
                x = disc_v[sl] * (a - diff_v[sl])
                diff_v[sl] = 1.0 / (1.0 + jnp.exp(-x))
            writes.append(pltpu.async_copy(
                diff_v.at[pl.ds(q * quarter, quarter)],
                out_hbm.at[pl.ds(base + q * quarter, quarter)], semo))
        for wr in writes:
            wr.wait()

    return k(skills, ability16, difficulty, discrimination)


def kernel(skills, ability_table, difficulty_table, discrimination_table):
    batch = skills.shape[0]
    if skills.dtype != jnp.int32:
        skills = skills.astype(jnp.int32)
    ability16 = jnp.broadcast_to(ability_table.reshape(()), (_LANES,))
    diff = difficulty_table.reshape(-1)
    disc = discrimination_table.reshape(-1)
    out = _irt_sc(skills, ability16, diff, disc, batch=batch)
    return out.reshape(batch, 1)


# speedup vs baseline: 3.2456x; 1.0094x over previous
import functools

import jax
import jax.numpy as jnp
from jax import lax
from jax.experimental import pallas as pl
from jax.experimental.pallas import tpu as pltpu
from jax.experimental.pallas import tpu_sc as plsc

_NC = 2
_NS = 16
_NW = _NC * _NS
_LANES = 16


@functools.partial(jax.jit, static_argnames=("batch",))
def _irt_sc(skills, ability16, difficulty, discrimination, *, batch):
    b_per_w = batch // _NW
    half = b_per_w // 2
    mesh = plsc.VectorSubcoreMesh(
        core_axis_name="c", subcore_axis_name="s",
        num_cores=_NC, num_subcores=_NS)

    @functools.partial(
        pl.kernel,
        out_type=jax.ShapeDtypeStruct((batch,), jnp.float32),
        mesh=mesh,
        scratch_types=[
            pltpu.VMEM((b_per_w,), jnp.int32),
            pltpu.VMEM((b_per_w,), jnp.float32),
            pltpu.VMEM((b_per_w,), jnp.float32),
            pltpu.VMEM((_LANES,), jnp.float32),
            pltpu.SemaphoreType.DMA,
            pltpu.SemaphoreType.DMA,
            pltpu.SemaphoreType.DMA,
        ],
    )
    def k(skills_hbm, ab_hbm, diff_hbm, disc_hbm, out_hbm,
          idx_v, diff_v, disc_v, ab_v, sem0, sem1, semo):
        wid = lax.axis_index("s") * _NC + lax.axis_index("c")
        base = wid * b_per_w
        lo = pl.ds(0, half)
        hi = pl.ds(half, half)
        pltpu.sync_copy(skills_hbm.at[pl.ds(base, half)], idx_v.at[lo])
        cp0 = [
            pltpu.async_copy(diff_hbm.at[idx_v.at[lo]], diff_v.at[lo], sem0),
            pltpu.async_copy(disc_hbm.at[idx_v.at[lo]], disc_v.at[lo], sem0),
        ]
        pltpu.sync_copy(skills_hbm.at[pl.ds(base + half, half)], idx_v.at[hi])
        cp1 = [
            pltpu.async_copy(diff_hbm.at[idx_v.at[hi]], diff_v.at[hi], sem1),
            pltpu.async_copy(disc_hbm.at[idx_v.at[hi]], disc_v.at[hi], sem1),
        ]
        pltpu.sync_copy(ab_hbm, ab_v)
        a = ab_v[:]
        for cp in cp0:
            cp.wait()
        for i in range(half // _LANES):
            sl = pl.ds(i * _LANES, _LANES)
            x = disc_v[sl] * (a - diff_v[sl])
            diff_v[sl] = 1.0 / (1.0 + jnp.exp(-x))
        wr0 = pltpu.async_copy(
            diff_v.at[lo], out_hbm.at[pl.ds(base, half)], semo)
        for cp in cp1:
            cp.wait()
        for i in range(half // _LANES, b_per_w // _LANES):
            sl = pl.ds(i * _LANES, _LANES)
            x = disc_v[sl] * (a - diff_v[sl])
            diff_v[sl] = 1.0 / (1.0 + jnp.exp(-x))
        wr1 = pltpu.async_copy(
            diff_v.at[hi], out_hbm.at[pl.ds(base + half, half)], semo)
        wr0.wait()
        wr1.wait()

    return k(skills, ability16, difficulty, discrimination)


def kernel(skills, ability_table, difficulty_table, discrimination_table):
    batch = skills.shape[0]
    if skills.dtype != jnp.int32:
        skills = skills.astype(jnp.int32)
    ability16 = jnp.broadcast_to(ability_table.reshape(()), (_LANES,))
    diff = difficulty_table.reshape(-1)
    disc = discrimination_table.reshape(-1)
    out = _irt_sc(skills, ability16, diff, disc, batch=batch)
    return out.reshape(batch, 1)
